# Initial kernel scaffold; baseline (speedup 1.0000x reference)
import jax
import jax.numpy as jnp
from jax import lax
from jax.experimental import pallas as pl
from jax.experimental.pallas import tpu as pltpu
from jax.experimental.pallas import tpu_sc as plsc

CHUNK = 128
NBUF = 2


def _make_gather(emb, n_total, nc, ns):
    nw = nc * ns
    rows_per_tile = n_total // nw
    nchunks = rows_per_tile // CHUNK
    mesh = plsc.VectorSubcoreMesh(core_axis_name="c", subcore_axis_name="s")

    def body(table_hbm, q_hbm, out_hbm, idx_v, rows_v, gsem0, gsem1):
        gsems = (gsem0, gsem1)
        wid = lax.axis_index("s") * nc + lax.axis_index("c")
        idx_row_base = wid * nchunks
        out_base = wid * rows_per_tile

        pltpu.sync_copy(q_hbm.at[pl.ds(idx_row_base, nchunks)], idx_v)

        for b in range(NBUF):
            pltpu.async_copy(table_hbm.at[idx_v.at[b]], rows_v.at[b], gsems[b])

        @pl.loop(0, nchunks - NBUF, step=NBUF)
        def _(j):
            for b in range(NBUF):
                cur = j + b
                pltpu.make_async_copy(
                    table_hbm.at[idx_v.at[b]], rows_v.at[b], gsems[b]
                ).wait()
                pltpu.sync_copy(
                    rows_v.at[b],
                    out_hbm.at[pl.ds(out_base + cur * CHUNK, CHUNK)],
                )
                pltpu.async_copy(
                    table_hbm.at[idx_v.at[cur + NBUF]], rows_v.at[b], gsems[b]
                )

        for b in range(NBUF):
            cur = nchunks - NBUF + b
            pltpu.make_async_copy(
                table_hbm.at[idx_v.at[b]], rows_v.at[b], gsems[b]
            ).wait()
            pltpu.sync_copy(
                rows_v.at[b],
                out_hbm.at[pl.ds(out_base + cur * CHUNK, CHUNK)],
            )

    return pl.kernel(
        body,
        out_type=jax.ShapeDtypeStruct((n_total, emb), jnp.float32),
        mesh=mesh,
        scratch_types=[
            pltpu.VMEM((nchunks, CHUNK), jnp.int32),
            pltpu.VMEM((NBUF, CHUNK, emb), jnp.float32),
            pltpu.SemaphoreType.DMA,
            pltpu.SemaphoreType.DMA,
        ],
    )


def kernel(q, table):
    batch, hist = q.shape
    _, emb = table.shape
    n_total = batch * hist
    info = plsc.get_sparse_core_info()
    nc, ns = info.num_cores, info.num_subcores
    qf = q.astype(jnp.int32).reshape(n_total // CHUNK, CHUNK)
    out = _make_gather(emb, n_total, nc, ns)(table, qf)
    return out.reshape(batch, hist, emb)


# speedup vs baseline: 1.2789x; 1.2789x over previous
import jax
import jax.numpy as jnp
from jax import lax
from jax.experimental import pallas as pl
from jax.experimental.pallas import tpu as pltpu
from jax.experimental.pallas import tpu_sc as plsc

CHUNK = 128
NBUF = 2


def _make_gather(emb, n_total, nc, ns):
    nw = nc * ns
    rows_per_tile = n_total // nw
    nchunks = rows_per_tile // CHUNK
    mesh = plsc.VectorSubcoreMesh(core_axis_name="c", subcore_axis_name="s")

    def body(table_hbm, q_hbm, out_hbm, idx_v, rows_v, gsem0, gsem1):
        gsems = (gsem0, gsem1)
        wid = lax.axis_index("s") * nc + lax.axis_index("c")
        out_base = wid * rows_per_tile

        pltpu.sync_copy(q_hbm.at[pl.ds(out_base, rows_per_tile)], idx_v)

        for b in range(NBUF):
            pltpu.async_copy(
                table_hbm.at[idx_v.at[pl.ds(b * CHUNK, CHUNK)]],
                rows_v.at[b], gsems[b],
            )

        @pl.loop(0, nchunks - NBUF, step=NBUF)
        def _(j):
            for b in range(NBUF):
                cur = j + b
                pltpu.make_async_copy(
                    table_hbm.at[idx_v.at[pl.ds(0, CHUNK)]], rows_v.at[b], gsems[b]
                ).wait()
                pltpu.sync_copy(
                    rows_v.at[b],
                    out_hbm.at[pl.ds(out_base + cur * CHUNK, CHUNK)],
                )
                pltpu.async_copy(
                    table_hbm.at[idx_v.at[pl.ds((cur + NBUF) * CHUNK, CHUNK)]],
                    rows_v.at[b], gsems[b],
                )

        for b in range(NBUF):
            cur = nchunks - NBUF + b
            pltpu.make_async_copy(
                table_hbm.at[idx_v.at[pl.ds(0, CHUNK)]], rows_v.at[b], gsems[b]
            ).wait()
            pltpu.sync_copy(
                rows_v.at[b],
                out_hbm.at[pl.ds(out_base + cur * CHUNK, CHUNK)],
            )

    return pl.kernel(
        body,
        out_type=jax.ShapeDtypeStruct((n_total, emb), jnp.float32),
        mesh=mesh,
        scratch_types=[
            pltpu.VMEM((rows_per_tile,), jnp.int32),
            pltpu.VMEM((NBUF, CHUNK, emb), jnp.float32),
            pltpu.SemaphoreType.DMA,
            pltpu.SemaphoreType.DMA,
        ],
    )


def kernel(q, table):
    batch, hist = q.shape
    _, emb = table.shape
    n_total = batch * hist
    info = plsc.get_sparse_core_info()
    nc, ns = info.num_cores, info.num_subcores
    qf = q.astype(jnp.int32).reshape(n_total)
    out = _make_gather(emb, n_total, nc, ns)(table, qf)
    return out.reshape(batch, hist, emb)
